# baseline (speedup 1.0000x reference)
import jax
import jax.numpy as jnp
from jax.experimental import pallas as pl


def _gcn_dense(agg, Xc, pW, sW, b, relu=True):
    N_, Din = Xc.shape
    Dout = pW.shape[1]
    BN = 1000

    def body(agg_ref, x_ref, pw_ref, sw_ref, b_ref, o_ref):
        acc = jnp.dot(agg_ref[...], pw_ref[...], preferred_element_type=jnp.float32)
        acc = acc + jnp.dot(x_ref[...], sw_ref[...], preferred_element_type=jnp.float32)
        acc = acc + b_ref[...]
        if relu:
            acc = jnp.maximum(acc, 0.0)
        o_ref[...] = acc

    return pl.pallas_call(
        body,
        grid=(N_ // BN,),
        in_specs=[
            pl.BlockSpec((BN, Din), lambda i: (i, 0)),
            pl.BlockSpec((BN, Din), lambda i: (i, 0)),
            pl.BlockSpec((Din, Dout), lambda i: (0, 0)),
            pl.BlockSpec((Din, Dout), lambda i: (0, 0)),
            pl.BlockSpec((1, Dout), lambda i: (0, 0)),
        ],
        out_specs=pl.BlockSpec((BN, Dout), lambda i: (i, 0)),
        out_shape=jax.ShapeDtypeStruct((N_, Dout), jnp.float32),
    )(agg, Xc, pW, sW, b.reshape(1, -1))


def _edge_head(PR, QC, w2row, b2):
    E_ = PR.shape[0]
    H_ = PR.shape[1]
    BE = 512

    def body(pr_ref, qc_ref, w2_ref, b2_ref, o_ref):
        h = jnp.maximum(pr_ref[...] + qc_ref[...], 0.0)
        s = jnp.sum(h * w2_ref[...], axis=1) + b2_ref[0, 0]
        o_ref[...] = jax.nn.sigmoid(s)

    return pl.pallas_call(
        body,
        grid=(E_ // BE,),
        in_specs=[
            pl.BlockSpec((BE, H_), lambda i: (i, 0)),
            pl.BlockSpec((BE, H_), lambda i: (i, 0)),
            pl.BlockSpec((1, H_), lambda i: (0, 0)),
            pl.BlockSpec((1, 1), lambda i: (0, 0)),
        ],
        out_specs=pl.BlockSpec((BE,), lambda i: (i,)),
        out_shape=jax.ShapeDtypeStruct((E_,), jnp.float32),
    )(PR, QC, w2row, b2.reshape(1, 1))


def kernel(X, edge_index, edge_weight,
           pass_W1, pass_b1, self_W1, self_b1,
           pass_W2, pass_b2, self_W2, self_b2,
           lin_W1, lin_b1, lin_W2, lin_b2):
    N_ = X.shape[0]
    H_ = pass_W1.shape[1]
    row = edge_index[0]
    col = edge_index[1]

    msgs1 = edge_weight[:, None] * jnp.take(X, col, axis=0)
    agg1 = jax.ops.segment_sum(msgs1, row, num_segments=N_)
    X1 = _gcn_dense(agg1, X, pass_W1, self_W1, pass_b1 + self_b1)

    msgs2 = edge_weight[:, None] * jnp.take(X1, col, axis=0)
    agg2 = jax.ops.segment_sum(msgs2, row, num_segments=N_)
    X2 = _gcn_dense(agg2, X1, pass_W2, self_W2, pass_b2 + self_b2)

    W_pq = jnp.concatenate([lin_W1[:H_], lin_W1[H_:]], axis=1)
    b_pq = jnp.concatenate([lin_b1, jnp.zeros_like(lin_b1)])

    BN = 1000

    def pq_body(x_ref, w_ref, b_ref, o_ref):
        o_ref[...] = jnp.dot(x_ref[...], w_ref[...], preferred_element_type=jnp.float32) + b_ref[...]

    PQmat = pl.pallas_call(
        pq_body,
        grid=(N_ // BN,),
        in_specs=[
            pl.BlockSpec((BN, H_), lambda i: (i, 0)),
            pl.BlockSpec((H_, 2 * H_), lambda i: (0, 0)),
            pl.BlockSpec((1, 2 * H_), lambda i: (0, 0)),
        ],
        out_specs=pl.BlockSpec((BN, 2 * H_), lambda i: (i, 0)),
        out_shape=jax.ShapeDtypeStruct((N_, 2 * H_), jnp.float32),
    )(X2, W_pq, b_pq.reshape(1, -1))
    P = PQmat[:, :H_]
    Q = PQmat[:, H_:]

    PR = jnp.take(P, row, axis=0)
    QC = jnp.take(Q, col, axis=0)
    return _edge_head(PR, QC, lin_W2[:, 0].reshape(1, -1), lin_b2[0])
